# baseline (speedup 1.0000x reference)
import functools

import jax
import jax.numpy as jnp
from jax import lax
from jax.experimental import pallas as pl
from jax.experimental.pallas import tpu as pltpu
from jax.experimental.pallas import tpu_sc as plsc

N = 10000
NP = 10240
E = 320000
D = 128
D_EDGE = 16

NUM_SC = 2
NUM_TILES = 16
CHUNK = 64
CHUNKS_PER_TILE = 320
GRP = 32
EDGES_PER_TILE = CHUNKS_PER_TILE * CHUNK
EP = NUM_TILES * EDGES_PER_TILE
IDX_ROWS = NUM_TILES * CHUNKS_PER_TILE
ROWS_PER_TILE = NP // NUM_TILES

EL_BLK = 512
EL_REAL_BLOCKS = E // EL_BLK
EL_BLOCKS = EP // EL_BLK

MLP_BLK = 2048
NEG_BIG = -1.0e30

import numpy as _np
PERM_LO = _np.empty((D // 2,), dtype=_np.int32)
PERM_HI = _np.empty((D // 2,), dtype=_np.int32)
for _g in range(D // 32):
    for _k in range(16):
        PERM_LO[16 * _g + _k] = 32 * _g + _k
        PERM_HI[16 * _g + _k] = 32 * _g + 16 + _k


def _edge_lin_body(ea_ref, wa_ref, ba_ref, wb_ref, bb_ref, out_ref):
    i = pl.program_id(0)
    ea = ea_ref[...]
    va = jnp.dot(ea, wa_ref[...], preferred_element_type=jnp.float32) + ba_ref[...]
    vb = jnp.dot(ea, wb_ref[...], preferred_element_type=jnp.float32) + bb_ref[...]
    va = jnp.where(i >= EL_REAL_BLOCKS, jnp.full_like(va, NEG_BIG), va)
    vb = jnp.where(i >= EL_REAL_BLOCKS, jnp.full_like(vb, NEG_BIG), vb)
    a16 = lax.bitcast_convert_type(va.astype(jnp.bfloat16), jnp.uint16)
    b16 = lax.bitcast_convert_type(vb.astype(jnp.bfloat16), jnp.uint16)
    packed = (a16.astype(jnp.uint32)
              | lax.shift_left(b16.astype(jnp.uint32), jnp.uint32(16)))
    out_ref[...] = lax.bitcast_convert_type(packed, jnp.int32)


def _edge_lin(ea_pad, we, be):
    wa = we[:, PERM_LO]
    wb = we[:, PERM_HI]
    ba = be[PERM_LO].reshape(1, D // 2)
    bb = be[PERM_HI].reshape(1, D // 2)
    return pl.pallas_call(
        _edge_lin_body,
        grid=(EL_BLOCKS,),
        in_specs=[
            pl.BlockSpec((EL_BLK, D_EDGE), lambda i: (i, 0)),
            pl.BlockSpec((D_EDGE, D // 2), lambda i: (0, 0)),
            pl.BlockSpec((1, D // 2), lambda i: (0, 0)),
            pl.BlockSpec((D_EDGE, D // 2), lambda i: (0, 0)),
            pl.BlockSpec((1, D // 2), lambda i: (0, 0)),
        ],
        out_specs=pl.BlockSpec((EL_BLK, D // 2), lambda i: (i, 0)),
        out_shape=jax.ShapeDtypeStruct((EP, D // 2), jnp.int32),
    )(ea_pad, wa, ba, wb, bb)


def _pack16(va, vb):
    a16 = lax.bitcast_convert_type(va.astype(jnp.bfloat16), jnp.uint16)
    b16 = lax.bitcast_convert_type(vb.astype(jnp.bfloat16), jnp.uint16)
    packed = (a16.astype(jnp.uint32)
              | lax.shift_left(b16.astype(jnp.uint32), jnp.uint32(16)))
    return lax.bitcast_convert_type(packed, jnp.int32)


def _mlp_body(h_ref, agg_ref, w1_ref, b1_ref, w2_ref, b2_ref, eps_ref, out_ref,
              *, out_relu):
    u = (1.0 + eps_ref[0, 0]) * h_ref[...] + agg_ref[...]
    t = jnp.maximum(jnp.dot(u, w1_ref[...], preferred_element_type=jnp.float32)
                    + b1_ref[...], 0.0)
    y = jnp.dot(t, w2_ref[...], preferred_element_type=jnp.float32) + b2_ref[...]
    if out_relu:
        y = jnp.maximum(y, 0.0)
    out_ref[...] = y


def _mlp(h, agg, w1, b1, w2, b2, eps, out_relu):
    nb = (2 * NP) // MLP_BLK
    return pl.pallas_call(
        functools.partial(_mlp_body, out_relu=out_relu),
        grid=(nb,),
        in_specs=[
            pl.BlockSpec((MLP_BLK, D), lambda i: (i, 0)),
            pl.BlockSpec((MLP_BLK, D), lambda i: (i, 0)),
            pl.BlockSpec((D, D), lambda i: (0, 0)),
            pl.BlockSpec((1, D), lambda i: (0, 0)),
            pl.BlockSpec((D, D), lambda i: (0, 0)),
            pl.BlockSpec((1, D), lambda i: (0, 0)),
            pl.BlockSpec((1, 1), lambda i: (0, 0)),
        ],
        out_specs=pl.BlockSpec((MLP_BLK, D), lambda i: (i, 0)),
        out_shape=jax.ShapeDtypeStruct((2 * NP, D), jnp.float32),
    )(h, agg, w1, b1, w2, b2, eps)


def _rho_body(yp_ref, yn_ref, w1_ref, b1_ref, w2_ref, b2_ref, out_ref):
    z = yp_ref[...] + yn_ref[...]
    t = jnp.maximum(jnp.dot(z, w1_ref[...], preferred_element_type=jnp.float32)
                    + b1_ref[...], 0.0)
    out_ref[...] = jnp.dot(t, w2_ref[...], preferred_element_type=jnp.float32) \
        + b2_ref[...]


def _rho(y, w1, b1, w2, b2):
    nb = NP // MLP_BLK
    return pl.pallas_call(
        _rho_body,
        grid=(nb,),
        in_specs=[
            pl.BlockSpec((MLP_BLK, D), lambda i: (i, 0)),
            pl.BlockSpec((MLP_BLK, D), lambda i: (i + nb, 0)),
            pl.BlockSpec((D, D), lambda i: (0, 0)),
            pl.BlockSpec((1, D), lambda i: (0, 0)),
            pl.BlockSpec((D, D), lambda i: (0, 0)),
            pl.BlockSpec((1, D), lambda i: (0, 0)),
        ],
        out_specs=pl.BlockSpec((MLP_BLK, D), lambda i: (i, 0)),
        out_shape=jax.ShapeDtypeStruct((NP, D), jnp.float32),
    )(y, y, w1, b1, w2, b2)


def _sc_body(hcat_hbm, e_hbm, srccat_hbm, dst_hbm, zeros_hbm, out_hbm,
             src_v, dst_v, hbuf, ebuf, acc,
             gsem, esem, ssem, isrc, idst):
    c = lax.axis_index("c")
    s = lax.axis_index("s")
    n_grps = CHUNKS_PER_TILE // GRP

    pltpu.sync_copy(zeros_hbm.at[pl.ds(s * ROWS_PER_TILE, ROWS_PER_TILE)],
                    acc.at[pl.ds(s * ROWS_PER_TILE, ROWS_PER_TILE)])
    plsc.subcore_barrier()

    def stage_start(grp, gslot):
        base_row = s * CHUNKS_PER_TILE + grp * GRP
        pltpu.async_copy(srccat_hbm.at[pl.ds(c * IDX_ROWS + base_row, GRP)],
                         src_v.at[gslot], isrc.at[gslot])
        pltpu.async_copy(dst_hbm.at[pl.ds(base_row, GRP)],
                         dst_v.at[gslot], idst.at[gslot])

    def stage_wait(grp, gslot):
        base_row = s * CHUNKS_PER_TILE + grp * GRP
        pltpu.make_async_copy(srccat_hbm.at[pl.ds(c * IDX_ROWS + base_row, GRP)],
                              src_v.at[gslot], isrc.at[gslot]).wait()
        pltpu.make_async_copy(dst_hbm.at[pl.ds(base_row, GRP)],
                              dst_v.at[gslot], idst.at[gslot]).wait()

    def gstart(gslot, n, slot):
        pltpu.async_copy(hcat_hbm.at[src_v.at[gslot, n]], hbuf.at[slot],
                         gsem.at[slot])

    def gwait(gslot, n, slot):
        pltpu.make_async_copy(hcat_hbm.at[src_v.at[gslot, n]], hbuf.at[slot],
                              gsem.at[slot]).wait()

    def estart(grp, n, slot):
        ebase = (s * CHUNKS_PER_TILE + grp * GRP + n) * CHUNK
        pltpu.async_copy(e_hbm.at[pl.ds(ebase, CHUNK)], ebuf.at[slot],
                         esem.at[slot])

    def ewait(grp, n, slot):
        ebase = (s * CHUNKS_PER_TILE + grp * GRP + n) * CHUNK
        pltpu.make_async_copy(e_hbm.at[pl.ds(ebase, CHUNK)], ebuf.at[slot],
                              esem.at[slot]).wait()

    def scat_wait(gslot, n, slot):
        pltpu.make_async_copy(hbuf.at[slot], acc.at[dst_v.at[gslot, n]],
                              ssem.at[slot]).wait()

    def compute(slot, eslot):
        hb = hbuf.at[slot]
        eb = ebuf.at[eslot]

        def row_body(r4, rc):
            for k in range(4):
                r = 4 * r4 + k
                for g in range(D // 32):
                    vi = eb[r, pl.ds(g * 16, 16)]
                    lo = lax.bitcast_convert_type(lax.shift_left(vi, 16),
                                                  jnp.float32)
                    hi = lax.bitcast_convert_type(
                        lax.bitwise_and(vi, jnp.int32(-65536)), jnp.float32)
                    sl_lo = pl.ds(g * 32, 16)
                    sl_hi = pl.ds(g * 32 + 16, 16)
                    hb[r, sl_lo] = jnp.maximum(hb[r, sl_lo] + lo, 0.0)
                    hb[r, sl_hi] = jnp.maximum(hb[r, sl_hi] + hi, 0.0)
            return rc

        lax.fori_loop(0, CHUNK // 4, row_body, 0)

    stage_start(0, 0)

    def group_pair_body(go, carry):
        for gslot in (0, 1):
            grp = 2 * go + gslot
            stage_wait(grp, gslot)

            @pl.when(grp + 1 < n_grps)
            def _():
                stage_start(grp + 1, gslot ^ 1)

            gstart(gslot, 0, 0)
            estart(grp, 0, 0)

            def pair_body(i, pc):
                for b in (0, 1):
                    cur = 2 * i + b

                    @pl.when(cur + 1 < GRP)
                    def _():
                        if b == 1:
                            scat_wait(gslot, cur - 1, b ^ 1)
                        else:
                            @pl.when(cur >= 1)
                            def _():
                                scat_wait(gslot, cur - 1, b ^ 1)
                        gstart(gslot, cur + 1, b ^ 1)
                        estart(grp, cur + 1, b ^ 1)

                    gwait(gslot, cur, b)
                    ewait(grp, cur, b)
                    compute(b, b)
                    pltpu.async_copy(hbuf.at[b], acc.at[dst_v.at[gslot, cur]],
                                     ssem.at[b], add=True)
                return pc

            lax.fori_loop(0, GRP // 2, pair_body, 0)
            scat_wait(gslot, GRP - 2, 0)
            scat_wait(gslot, GRP - 1, 1)
        return carry

    lax.fori_loop(0, n_grps // 2, group_pair_body, 0)
    plsc.subcore_barrier()

    pltpu.sync_copy(acc.at[pl.ds(s * ROWS_PER_TILE, ROWS_PER_TILE)],
                    out_hbm.at[pl.ds(c * NP + s * ROWS_PER_TILE, ROWS_PER_TILE)])


@functools.partial(
    pl.kernel,
    mesh=plsc.VectorSubcoreMesh(core_axis_name="c", subcore_axis_name="s"),
    out_type=jax.ShapeDtypeStruct((2 * NP, D), jnp.float32),
    scratch_types=[
        pltpu.VMEM((2, GRP, CHUNK), jnp.int32),
        pltpu.VMEM((2, GRP, CHUNK), jnp.int32),
        pltpu.VMEM((2, CHUNK, D), jnp.float32),
        pltpu.VMEM((2, CHUNK, D // 2), jnp.int32),
        pltpu.VMEM_SHARED((NP, D), jnp.float32),
        pltpu.SemaphoreType.DMA((2,)),
        pltpu.SemaphoreType.DMA((2,)),
        pltpu.SemaphoreType.DMA((2,)),
        pltpu.SemaphoreType.DMA((2,)),
        pltpu.SemaphoreType.DMA((2,)),
    ],
)
def _sc_pass(hcat_hbm, e_hbm, srccat_hbm, dst_hbm, zeros_hbm, out_hbm,
             src_v, dst_v, hbuf, ebuf, acc, gsem, esem, ssem, isrc, idst):
    _sc_body(hcat_hbm, e_hbm, srccat_hbm, dst_hbm, zeros_hbm, out_hbm,
             src_v, dst_v, hbuf, ebuf, acc, gsem, esem, ssem, isrc, idst)


def kernel(eigen_vectors, edge_index, edge_attr, params):
    x = eigen_vectors
    src = edge_index[0]
    dst = edge_index[1]
    epad = EP - E

    src_p = jnp.concatenate([src, jnp.zeros((epad,), jnp.int32)])
    dst_p = jnp.concatenate([dst, jnp.zeros((epad,), jnp.int32)])
    srccat = jnp.concatenate([src_p, src_p + NP]).reshape(2 * IDX_ROWS, CHUNK)
    dst2d = dst_p.reshape(IDX_ROWS, CHUNK)
    ea_pad = jnp.concatenate(
        [edge_attr, jnp.zeros((epad, D_EDGE), jnp.float32)], axis=0)
    zeros = jnp.zeros((NP, D), jnp.float32)

    xp = jnp.concatenate([x, jnp.zeros((NP - N, D), jnp.float32)], axis=0)
    hcat = jnp.concatenate([xp, -xp], axis=0)
    layers = params['phi']
    e_all = [_edge_lin(ea_pad, p['We'], p['be']) for p in layers]
    for li, p in enumerate(layers):
        e = e_all[li]
        agg = _sc_pass(hcat, e, srccat, dst2d, zeros)
        hcat = _mlp(hcat, agg, p['W1'], p['b1'].reshape(1, D),
                    p['W2'], p['b2'].reshape(1, D),
                    p['eps'].reshape(1, 1), out_relu=(li < len(layers) - 1))

    r = params['rho']
    out = _rho(hcat, r['W1'], r['b1'].reshape(1, D),
               r['W2'], r['b2'].reshape(1, D))
    return out[:N]


# speedup vs baseline: 1.0047x; 1.0047x over previous
import functools

import jax
import jax.numpy as jnp
from jax import lax
from jax.experimental import pallas as pl
from jax.experimental.pallas import tpu as pltpu
from jax.experimental.pallas import tpu_sc as plsc

N = 10000
NP = 10240
E = 320000
D = 128
D_EDGE = 16

NUM_SC = 2
NUM_TILES = 16
CHUNK = 64
CHUNKS_PER_TILE = 320
GRP = 32
EDGES_PER_TILE = CHUNKS_PER_TILE * CHUNK
EP = NUM_TILES * EDGES_PER_TILE
IDX_ROWS = NUM_TILES * CHUNKS_PER_TILE
ROWS_PER_TILE = NP // NUM_TILES

EL_BLK = 512
EL_REAL_BLOCKS = E // EL_BLK
EL_BLOCKS = EP // EL_BLK

MLP_BLK = 2048
NEG_BIG = -1.0e30

import numpy as _np
PERM_LO = _np.empty((D // 2,), dtype=_np.int32)
PERM_HI = _np.empty((D // 2,), dtype=_np.int32)
for _g in range(D // 32):
    for _k in range(16):
        PERM_LO[16 * _g + _k] = 32 * _g + _k
        PERM_HI[16 * _g + _k] = 32 * _g + 16 + _k


def _edge_lin_body(ea_ref, wa_ref, ba_ref, wb_ref, bb_ref, out_ref):
    i = pl.program_id(0)
    ea = ea_ref[...]
    va = jnp.dot(ea, wa_ref[...], preferred_element_type=jnp.float32) + ba_ref[...]
    vb = jnp.dot(ea, wb_ref[...], preferred_element_type=jnp.float32) + bb_ref[...]
    va = jnp.where(i >= EL_REAL_BLOCKS, jnp.full_like(va, NEG_BIG), va)
    vb = jnp.where(i >= EL_REAL_BLOCKS, jnp.full_like(vb, NEG_BIG), vb)
    a16 = lax.bitcast_convert_type(va.astype(jnp.bfloat16), jnp.uint16)
    b16 = lax.bitcast_convert_type(vb.astype(jnp.bfloat16), jnp.uint16)
    packed = (a16.astype(jnp.uint32)
              | lax.shift_left(b16.astype(jnp.uint32), jnp.uint32(16)))
    out_ref[...] = lax.bitcast_convert_type(packed, jnp.int32)


def _edge_lin(ea_pad, we, be):
    wa = we[:, PERM_LO]
    wb = we[:, PERM_HI]
    ba = be[PERM_LO].reshape(1, D // 2)
    bb = be[PERM_HI].reshape(1, D // 2)
    return pl.pallas_call(
        _edge_lin_body,
        grid=(EL_BLOCKS,),
        in_specs=[
            pl.BlockSpec((EL_BLK, D_EDGE), lambda i: (i, 0)),
            pl.BlockSpec((D_EDGE, D // 2), lambda i: (0, 0)),
            pl.BlockSpec((1, D // 2), lambda i: (0, 0)),
            pl.BlockSpec((D_EDGE, D // 2), lambda i: (0, 0)),
            pl.BlockSpec((1, D // 2), lambda i: (0, 0)),
        ],
        out_specs=pl.BlockSpec((EL_BLK, D // 2), lambda i: (i, 0)),
        out_shape=jax.ShapeDtypeStruct((EP, D // 2), jnp.int32),
    )(ea_pad, wa, ba, wb, bb)


def _pack16(va, vb):
    a16 = lax.bitcast_convert_type(va.astype(jnp.bfloat16), jnp.uint16)
    b16 = lax.bitcast_convert_type(vb.astype(jnp.bfloat16), jnp.uint16)
    packed = (a16.astype(jnp.uint32)
              | lax.shift_left(b16.astype(jnp.uint32), jnp.uint32(16)))
    return lax.bitcast_convert_type(packed, jnp.int32)


def _mlp_body(h_ref, agg_ref, w1_ref, b1_ref, w2_ref, b2_ref, eps_ref, out_ref,
              *, out_relu):
    u = (1.0 + eps_ref[0, 0]) * h_ref[...] + agg_ref[...]
    t = jnp.maximum(jnp.dot(u, w1_ref[...], preferred_element_type=jnp.float32)
                    + b1_ref[...], 0.0)
    y = jnp.dot(t, w2_ref[...], preferred_element_type=jnp.float32) + b2_ref[...]
    if out_relu:
        y = jnp.maximum(y, 0.0)
    out_ref[...] = y


def _mlp(h, agg, w1, b1, w2, b2, eps, out_relu):
    nb = (2 * NP) // MLP_BLK
    return pl.pallas_call(
        functools.partial(_mlp_body, out_relu=out_relu),
        grid=(nb,),
        in_specs=[
            pl.BlockSpec((MLP_BLK, D), lambda i: (i, 0)),
            pl.BlockSpec((MLP_BLK, D), lambda i: (i, 0)),
            pl.BlockSpec((D, D), lambda i: (0, 0)),
            pl.BlockSpec((1, D), lambda i: (0, 0)),
            pl.BlockSpec((D, D), lambda i: (0, 0)),
            pl.BlockSpec((1, D), lambda i: (0, 0)),
            pl.BlockSpec((1, 1), lambda i: (0, 0)),
        ],
        out_specs=pl.BlockSpec((MLP_BLK, D), lambda i: (i, 0)),
        out_shape=jax.ShapeDtypeStruct((2 * NP, D), jnp.float32),
    )(h, agg, w1, b1, w2, b2, eps)


def _last_body(hp_ref, hn_ref, ap_ref, an_ref, w1_ref, b1_ref, w2_ref, b2_ref,
               eps_ref, rw1_ref, rb1_ref, rw2_ref, rb2_ref, out_ref):
    def gine(h, a):
        u = (1.0 + eps_ref[0, 0]) * h + a
        t = jnp.maximum(
            jnp.dot(u, w1_ref[...], preferred_element_type=jnp.float32)
            + b1_ref[...], 0.0)
        return jnp.dot(t, w2_ref[...],
                       preferred_element_type=jnp.float32) + b2_ref[...]

    z = gine(hp_ref[...], ap_ref[...]) + gine(hn_ref[...], an_ref[...])
    t = jnp.maximum(jnp.dot(z, rw1_ref[...], preferred_element_type=jnp.float32)
                    + rb1_ref[...], 0.0)
    out_ref[...] = jnp.dot(t, rw2_ref[...],
                           preferred_element_type=jnp.float32) + rb2_ref[...]


def _last_and_rho(h, agg, w1, b1, w2, b2, eps, rw1, rb1, rw2, rb2):
    nb = NP // MLP_BLK
    wspec = pl.BlockSpec((D, D), lambda i: (0, 0))
    bspec = pl.BlockSpec((1, D), lambda i: (0, 0))
    return pl.pallas_call(
        _last_body,
        grid=(nb,),
        in_specs=[
            pl.BlockSpec((MLP_BLK, D), lambda i: (i, 0)),
            pl.BlockSpec((MLP_BLK, D), lambda i: (i + nb, 0)),
            pl.BlockSpec((MLP_BLK, D), lambda i: (i, 0)),
            pl.BlockSpec((MLP_BLK, D), lambda i: (i + nb, 0)),
            wspec, bspec, wspec, bspec,
            pl.BlockSpec((1, 1), lambda i: (0, 0)),
            wspec, bspec, wspec, bspec,
        ],
        out_specs=pl.BlockSpec((MLP_BLK, D), lambda i: (i, 0)),
        out_shape=jax.ShapeDtypeStruct((NP, D), jnp.float32),
    )(h, h, agg, agg, w1, b1, w2, b2, eps, rw1, rb1, rw2, rb2)


def _sc_body(hcat_hbm, e_hbm, srccat_hbm, dst_hbm, zeros_hbm, out_hbm,
             src_v, dst_v, hbuf, ebuf, acc,
             gsem, esem, ssem, isrc, idst):
    c = lax.axis_index("c")
    s = lax.axis_index("s")
    n_grps = CHUNKS_PER_TILE // GRP

    pltpu.sync_copy(zeros_hbm.at[pl.ds(s * ROWS_PER_TILE, ROWS_PER_TILE)],
                    acc.at[pl.ds(s * ROWS_PER_TILE, ROWS_PER_TILE)])
    plsc.subcore_barrier()

    def stage_start(grp, gslot):
        base_row = s * CHUNKS_PER_TILE + grp * GRP
        pltpu.async_copy(srccat_hbm.at[pl.ds(c * IDX_ROWS + base_row, GRP)],
                         src_v.at[gslot], isrc.at[gslot])
        pltpu.async_copy(dst_hbm.at[pl.ds(base_row, GRP)],
                         dst_v.at[gslot], idst.at[gslot])

    def stage_wait(grp, gslot):
        base_row = s * CHUNKS_PER_TILE + grp * GRP
        pltpu.make_async_copy(srccat_hbm.at[pl.ds(c * IDX_ROWS + base_row, GRP)],
                              src_v.at[gslot], isrc.at[gslot]).wait()
        pltpu.make_async_copy(dst_hbm.at[pl.ds(base_row, GRP)],
                              dst_v.at[gslot], idst.at[gslot]).wait()

    def gstart(gslot, n, slot):
        pltpu.async_copy(hcat_hbm.at[src_v.at[gslot, n]], hbuf.at[slot],
                         gsem.at[slot])

    def gwait(gslot, n, slot):
        pltpu.make_async_copy(hcat_hbm.at[src_v.at[gslot, n]], hbuf.at[slot],
                              gsem.at[slot]).wait()

    def estart(grp, n, slot):
        ebase = (s * CHUNKS_PER_TILE + grp * GRP + n) * CHUNK
        pltpu.async_copy(e_hbm.at[pl.ds(ebase, CHUNK)], ebuf.at[slot],
                         esem.at[slot])

    def ewait(grp, n, slot):
        ebase = (s * CHUNKS_PER_TILE + grp * GRP + n) * CHUNK
        pltpu.make_async_copy(e_hbm.at[pl.ds(ebase, CHUNK)], ebuf.at[slot],
                              esem.at[slot]).wait()

    def scat_wait(gslot, n, slot):
        pltpu.make_async_copy(hbuf.at[slot], acc.at[dst_v.at[gslot, n]],
                              ssem.at[slot]).wait()

    def compute(slot, eslot):
        hb = hbuf.at[slot]
        eb = ebuf.at[eslot]

        def row_body(r4, rc):
            for k in range(4):
                r = 4 * r4 + k
                for g in range(D // 32):
                    vi = eb[r, pl.ds(g * 16, 16)]
                    lo = lax.bitcast_convert_type(lax.shift_left(vi, 16),
                                                  jnp.float32)
                    hi = lax.bitcast_convert_type(
                        lax.bitwise_and(vi, jnp.int32(-65536)), jnp.float32)
                    sl_lo = pl.ds(g * 32, 16)
                    sl_hi = pl.ds(g * 32 + 16, 16)
                    hb[r, sl_lo] = jnp.maximum(hb[r, sl_lo] + lo, 0.0)
                    hb[r, sl_hi] = jnp.maximum(hb[r, sl_hi] + hi, 0.0)
            return rc

        lax.fori_loop(0, CHUNK // 4, row_body, 0)

    stage_start(0, 0)

    def group_pair_body(go, carry):
        for gslot in (0, 1):
            grp = 2 * go + gslot
            stage_wait(grp, gslot)

            @pl.when(grp + 1 < n_grps)
            def _():
                stage_start(grp + 1, gslot ^ 1)

            gstart(gslot, 0, 0)
            estart(grp, 0, 0)

            def pair_body(i, pc):
                for b in (0, 1):
                    cur = 2 * i + b

                    @pl.when(cur + 1 < GRP)
                    def _():
                        if b == 1:
                            scat_wait(gslot, cur - 1, b ^ 1)
                        else:
                            @pl.when(cur >= 1)
                            def _():
                                scat_wait(gslot, cur - 1, b ^ 1)
                        gstart(gslot, cur + 1, b ^ 1)
                        estart(grp, cur + 1, b ^ 1)

                    gwait(gslot, cur, b)
                    ewait(grp, cur, b)
                    compute(b, b)
                    pltpu.async_copy(hbuf.at[b], acc.at[dst_v.at[gslot, cur]],
                                     ssem.at[b], add=True)
                return pc

            lax.fori_loop(0, GRP // 2, pair_body, 0)
            scat_wait(gslot, GRP - 2, 0)
            scat_wait(gslot, GRP - 1, 1)
        return carry

    lax.fori_loop(0, n_grps // 2, group_pair_body, 0)
    plsc.subcore_barrier()

    pltpu.sync_copy(acc.at[pl.ds(s * ROWS_PER_TILE, ROWS_PER_TILE)],
                    out_hbm.at[pl.ds(c * NP + s * ROWS_PER_TILE, ROWS_PER_TILE)])


@functools.partial(
    pl.kernel,
    mesh=plsc.VectorSubcoreMesh(core_axis_name="c", subcore_axis_name="s"),
    out_type=jax.ShapeDtypeStruct((2 * NP, D), jnp.float32),
    scratch_types=[
        pltpu.VMEM((2, GRP, CHUNK), jnp.int32),
        pltpu.VMEM((2, GRP, CHUNK), jnp.int32),
        pltpu.VMEM((2, CHUNK, D), jnp.float32),
        pltpu.VMEM((2, CHUNK, D // 2), jnp.int32),
        pltpu.VMEM_SHARED((NP, D), jnp.float32),
        pltpu.SemaphoreType.DMA((2,)),
        pltpu.SemaphoreType.DMA((2,)),
        pltpu.SemaphoreType.DMA((2,)),
        pltpu.SemaphoreType.DMA((2,)),
        pltpu.SemaphoreType.DMA((2,)),
    ],
)
def _sc_pass(hcat_hbm, e_hbm, srccat_hbm, dst_hbm, zeros_hbm, out_hbm,
             src_v, dst_v, hbuf, ebuf, acc, gsem, esem, ssem, isrc, idst):
    _sc_body(hcat_hbm, e_hbm, srccat_hbm, dst_hbm, zeros_hbm, out_hbm,
             src_v, dst_v, hbuf, ebuf, acc, gsem, esem, ssem, isrc, idst)


def kernel(eigen_vectors, edge_index, edge_attr, params):
    x = eigen_vectors
    src = edge_index[0]
    dst = edge_index[1]
    epad = EP - E

    src_p = jnp.concatenate([src, jnp.zeros((epad,), jnp.int32)])
    dst_p = jnp.concatenate([dst, jnp.zeros((epad,), jnp.int32)])
    srccat = jnp.concatenate([src_p, src_p + NP]).reshape(2 * IDX_ROWS, CHUNK)
    dst2d = dst_p.reshape(IDX_ROWS, CHUNK)
    ea_pad = jnp.concatenate(
        [edge_attr, jnp.zeros((epad, D_EDGE), jnp.float32)], axis=0)
    zeros = jnp.zeros((NP, D), jnp.float32)

    xp = jnp.concatenate([x, jnp.zeros((NP - N, D), jnp.float32)], axis=0)
    hcat = jnp.concatenate([xp, -xp], axis=0)
    layers = params['phi']
    e_all = [_edge_lin(ea_pad, p['We'], p['be']) for p in layers]
    for li, p in enumerate(layers[:-1]):
        agg = _sc_pass(hcat, e_all[li], srccat, dst2d, zeros)
        hcat = _mlp(hcat, agg, p['W1'], p['b1'].reshape(1, D),
                    p['W2'], p['b2'].reshape(1, D),
                    p['eps'].reshape(1, 1), out_relu=True)

    p = layers[-1]
    r = params['rho']
    agg = _sc_pass(hcat, e_all[-1], srccat, dst2d, zeros)
    out = _last_and_rho(hcat, agg, p['W1'], p['b1'].reshape(1, D),
                        p['W2'], p['b2'].reshape(1, D), p['eps'].reshape(1, 1),
                        r['W1'], r['b1'].reshape(1, D),
                        r['W2'], r['b2'].reshape(1, D))
    return out[:N]


# speedup vs baseline: 1.0056x; 1.0010x over previous
import functools

import jax
import jax.numpy as jnp
from jax import lax
from jax.experimental import pallas as pl
from jax.experimental.pallas import tpu as pltpu
from jax.experimental.pallas import tpu_sc as plsc

N = 10000
NP = 10240
E = 320000
D = 128
D_EDGE = 16

NUM_SC = 2
NUM_TILES = 16
CHUNK = 64
CHUNKS_PER_TILE = 320
GRP = 32
EDGES_PER_TILE = CHUNKS_PER_TILE * CHUNK
EP = NUM_TILES * EDGES_PER_TILE
IDX_ROWS = NUM_TILES * CHUNKS_PER_TILE
ROWS_PER_TILE = NP // NUM_TILES

EL_BLK = 512
EL_REAL_BLOCKS = E // EL_BLK
EL_BLOCKS = EP // EL_BLK

MLP_BLK = 2048
NEG_BIG = -1.0e30

import numpy as _np
PERM_LO = _np.empty((D // 2,), dtype=_np.int32)
PERM_HI = _np.empty((D // 2,), dtype=_np.int32)
for _g in range(D // 32):
    for _k in range(16):
        PERM_LO[16 * _g + _k] = 32 * _g + _k
        PERM_HI[16 * _g + _k] = 32 * _g + 16 + _k


def _edge_lin_body(ea_ref, wa_ref, ba_ref, wb_ref, bb_ref, out_ref):
    i = pl.program_id(0)
    ea = ea_ref[...]
    va = jnp.dot(ea, wa_ref[...], preferred_element_type=jnp.float32) + ba_ref[...]
    vb = jnp.dot(ea, wb_ref[...], preferred_element_type=jnp.float32) + bb_ref[...]
    va = jnp.where(i >= EL_REAL_BLOCKS, jnp.full_like(va, NEG_BIG), va)
    vb = jnp.where(i >= EL_REAL_BLOCKS, jnp.full_like(vb, NEG_BIG), vb)
    a16 = lax.bitcast_convert_type(va.astype(jnp.bfloat16), jnp.uint16)
    b16 = lax.bitcast_convert_type(vb.astype(jnp.bfloat16), jnp.uint16)
    packed = (a16.astype(jnp.uint32)
              | lax.shift_left(b16.astype(jnp.uint32), jnp.uint32(16)))
    out_ref[...] = lax.bitcast_convert_type(packed, jnp.int32)


def _edge_lin(ea_pad, we, be):
    wa = we[:, PERM_LO]
    wb = we[:, PERM_HI]
    ba = be[PERM_LO].reshape(1, D // 2)
    bb = be[PERM_HI].reshape(1, D // 2)
    return pl.pallas_call(
        _edge_lin_body,
        grid=(EL_BLOCKS,),
        in_specs=[
            pl.BlockSpec((EL_BLK, D_EDGE), lambda i: (i, 0)),
            pl.BlockSpec((D_EDGE, D // 2), lambda i: (0, 0)),
            pl.BlockSpec((1, D // 2), lambda i: (0, 0)),
            pl.BlockSpec((D_EDGE, D // 2), lambda i: (0, 0)),
            pl.BlockSpec((1, D // 2), lambda i: (0, 0)),
        ],
        out_specs=pl.BlockSpec((EL_BLK, D // 2), lambda i: (i, 0)),
        out_shape=jax.ShapeDtypeStruct((EP, D // 2), jnp.int32),
    )(ea_pad, wa, ba, wb, bb)


def _pack16(va, vb):
    a16 = lax.bitcast_convert_type(va.astype(jnp.bfloat16), jnp.uint16)
    b16 = lax.bitcast_convert_type(vb.astype(jnp.bfloat16), jnp.uint16)
    packed = (a16.astype(jnp.uint32)
              | lax.shift_left(b16.astype(jnp.uint32), jnp.uint32(16)))
    return lax.bitcast_convert_type(packed, jnp.int32)


def _mlp_body(h_ref, agg_ref, w1_ref, b1_ref, w2_ref, b2_ref, eps_ref, out_ref,
              *, out_relu):
    u = (1.0 + eps_ref[0, 0]) * h_ref[...] + agg_ref[...]
    t = jnp.maximum(jnp.dot(u, w1_ref[...], preferred_element_type=jnp.float32)
                    + b1_ref[...], 0.0)
    y = jnp.dot(t, w2_ref[...], preferred_element_type=jnp.float32) + b2_ref[...]
    if out_relu:
        y = jnp.maximum(y, 0.0)
    out_ref[...] = y


def _mlp(h, agg, w1, b1, w2, b2, eps, out_relu):
    blk = 2 * MLP_BLK
    nb = (2 * NP) // blk
    return pl.pallas_call(
        functools.partial(_mlp_body, out_relu=out_relu),
        grid=(nb,),
        in_specs=[
            pl.BlockSpec((blk, D), lambda i: (i, 0)),
            pl.BlockSpec((blk, D), lambda i: (i, 0)),
            pl.BlockSpec((D, D), lambda i: (0, 0)),
            pl.BlockSpec((1, D), lambda i: (0, 0)),
            pl.BlockSpec((D, D), lambda i: (0, 0)),
            pl.BlockSpec((1, D), lambda i: (0, 0)),
            pl.BlockSpec((1, 1), lambda i: (0, 0)),
        ],
        out_specs=pl.BlockSpec((blk, D), lambda i: (i, 0)),
        out_shape=jax.ShapeDtypeStruct((2 * NP, D), jnp.float32),
    )(h, agg, w1, b1, w2, b2, eps)


def _last_body(hp_ref, hn_ref, ap_ref, an_ref, w1_ref, b1_ref, w2_ref, b2_ref,
               eps_ref, rw1_ref, rb1_ref, rw2_ref, rb2_ref, out_ref):
    def gine(h, a):
        u = (1.0 + eps_ref[0, 0]) * h + a
        t = jnp.maximum(
            jnp.dot(u, w1_ref[...], preferred_element_type=jnp.float32)
            + b1_ref[...], 0.0)
        return jnp.dot(t, w2_ref[...],
                       preferred_element_type=jnp.float32) + b2_ref[...]

    z = gine(hp_ref[...], ap_ref[...]) + gine(hn_ref[...], an_ref[...])
    t = jnp.maximum(jnp.dot(z, rw1_ref[...], preferred_element_type=jnp.float32)
                    + rb1_ref[...], 0.0)
    out_ref[...] = jnp.dot(t, rw2_ref[...],
                           preferred_element_type=jnp.float32) + rb2_ref[...]


def _last_and_rho(h, agg, w1, b1, w2, b2, eps, rw1, rb1, rw2, rb2):
    nb = NP // MLP_BLK
    wspec = pl.BlockSpec((D, D), lambda i: (0, 0))
    bspec = pl.BlockSpec((1, D), lambda i: (0, 0))
    return pl.pallas_call(
        _last_body,
        grid=(nb,),
        in_specs=[
            pl.BlockSpec((MLP_BLK, D), lambda i: (i, 0)),
            pl.BlockSpec((MLP_BLK, D), lambda i: (i + nb, 0)),
            pl.BlockSpec((MLP_BLK, D), lambda i: (i, 0)),
            pl.BlockSpec((MLP_BLK, D), lambda i: (i + nb, 0)),
            wspec, bspec, wspec, bspec,
            pl.BlockSpec((1, 1), lambda i: (0, 0)),
            wspec, bspec, wspec, bspec,
        ],
        out_specs=pl.BlockSpec((MLP_BLK, D), lambda i: (i, 0)),
        out_shape=jax.ShapeDtypeStruct((NP, D), jnp.float32),
    )(h, h, agg, agg, w1, b1, w2, b2, eps, rw1, rb1, rw2, rb2)


def _sc_body(hcat_hbm, e_hbm, srccat_hbm, dst_hbm, zeros_hbm, out_hbm,
             src_v, dst_v, hbuf, ebuf, acc,
             gsem, esem, ssem, isrc, idst):
    c = lax.axis_index("c")
    s = lax.axis_index("s")
    n_grps = CHUNKS_PER_TILE // GRP

    pltpu.sync_copy(zeros_hbm.at[pl.ds(s * ROWS_PER_TILE, ROWS_PER_TILE)],
                    acc.at[pl.ds(s * ROWS_PER_TILE, ROWS_PER_TILE)])
    plsc.subcore_barrier()

    def stage_start(grp, gslot):
        base_row = s * CHUNKS_PER_TILE + grp * GRP
        pltpu.async_copy(srccat_hbm.at[pl.ds(c * IDX_ROWS + base_row, GRP)],
                         src_v.at[gslot], isrc.at[gslot])
        pltpu.async_copy(dst_hbm.at[pl.ds(base_row, GRP)],
                         dst_v.at[gslot], idst.at[gslot])

    def stage_wait(grp, gslot):
        base_row = s * CHUNKS_PER_TILE + grp * GRP
        pltpu.make_async_copy(srccat_hbm.at[pl.ds(c * IDX_ROWS + base_row, GRP)],
                              src_v.at[gslot], isrc.at[gslot]).wait()
        pltpu.make_async_copy(dst_hbm.at[pl.ds(base_row, GRP)],
                              dst_v.at[gslot], idst.at[gslot]).wait()

    def gstart(gslot, n, slot):
        pltpu.async_copy(hcat_hbm.at[src_v.at[gslot, n]], hbuf.at[slot],
                         gsem.at[slot])

    def gwait(gslot, n, slot):
        pltpu.make_async_copy(hcat_hbm.at[src_v.at[gslot, n]], hbuf.at[slot],
                              gsem.at[slot]).wait()

    def estart(grp, n, slot):
        ebase = (s * CHUNKS_PER_TILE + grp * GRP + n) * CHUNK
        pltpu.async_copy(e_hbm.at[pl.ds(ebase, CHUNK)], ebuf.at[slot],
                         esem.at[slot])

    def ewait(grp, n, slot):
        ebase = (s * CHUNKS_PER_TILE + grp * GRP + n) * CHUNK
        pltpu.make_async_copy(e_hbm.at[pl.ds(ebase, CHUNK)], ebuf.at[slot],
                              esem.at[slot]).wait()

    def scat_wait(gslot, n, slot):
        pltpu.make_async_copy(hbuf.at[slot], acc.at[dst_v.at[gslot, n]],
                              ssem.at[slot]).wait()

    def compute(slot, eslot):
        hb = hbuf.at[slot]
        eb = ebuf.at[eslot]

        def row_body(r4, rc):
            for k in range(4):
                r = 4 * r4 + k
                for g in range(D // 32):
                    vi = eb[r, pl.ds(g * 16, 16)]
                    lo = lax.bitcast_convert_type(lax.shift_left(vi, 16),
                                                  jnp.float32)
                    hi = lax.bitcast_convert_type(
                        lax.bitwise_and(vi, jnp.int32(-65536)), jnp.float32)
                    sl_lo = pl.ds(g * 32, 16)
                    sl_hi = pl.ds(g * 32 + 16, 16)
                    hb[r, sl_lo] = jnp.maximum(hb[r, sl_lo] + lo, 0.0)
                    hb[r, sl_hi] = jnp.maximum(hb[r, sl_hi] + hi, 0.0)
            return rc

        lax.fori_loop(0, CHUNK // 4, row_body, 0)

    stage_start(0, 0)

    def group_pair_body(go, carry):
        for gslot in (0, 1):
            grp = 2 * go + gslot
            stage_wait(grp, gslot)

            @pl.when(grp + 1 < n_grps)
            def _():
                stage_start(grp + 1, gslot ^ 1)

            gstart(gslot, 0, 0)
            estart(grp, 0, 0)

            def pair_body(i, pc):
                for b in (0, 1):
                    cur = 2 * i + b

                    @pl.when(cur + 1 < GRP)
                    def _():
                        if b == 1:
                            scat_wait(gslot, cur - 1, b ^ 1)
                        else:
                            @pl.when(cur >= 1)
                            def _():
                                scat_wait(gslot, cur - 1, b ^ 1)
                        gstart(gslot, cur + 1, b ^ 1)
                        estart(grp, cur + 1, b ^ 1)

                    gwait(gslot, cur, b)
                    ewait(grp, cur, b)
                    compute(b, b)
                    pltpu.async_copy(hbuf.at[b], acc.at[dst_v.at[gslot, cur]],
                                     ssem.at[b], add=True)
                return pc

            lax.fori_loop(0, GRP // 2, pair_body, 0)
            scat_wait(gslot, GRP - 2, 0)
            scat_wait(gslot, GRP - 1, 1)
        return carry

    lax.fori_loop(0, n_grps // 2, group_pair_body, 0)
    plsc.subcore_barrier()

    pltpu.sync_copy(acc.at[pl.ds(s * ROWS_PER_TILE, ROWS_PER_TILE)],
                    out_hbm.at[pl.ds(c * NP + s * ROWS_PER_TILE, ROWS_PER_TILE)])


@functools.partial(
    pl.kernel,
    mesh=plsc.VectorSubcoreMesh(core_axis_name="c", subcore_axis_name="s"),
    out_type=jax.ShapeDtypeStruct((2 * NP, D), jnp.float32),
    scratch_types=[
        pltpu.VMEM((2, GRP, CHUNK), jnp.int32),
        pltpu.VMEM((2, GRP, CHUNK), jnp.int32),
        pltpu.VMEM((2, CHUNK, D), jnp.float32),
        pltpu.VMEM((2, CHUNK, D // 2), jnp.int32),
        pltpu.VMEM_SHARED((NP, D), jnp.float32),
        pltpu.SemaphoreType.DMA((2,)),
        pltpu.SemaphoreType.DMA((2,)),
        pltpu.SemaphoreType.DMA((2,)),
        pltpu.SemaphoreType.DMA((2,)),
        pltpu.SemaphoreType.DMA((2,)),
    ],
)
def _sc_pass(hcat_hbm, e_hbm, srccat_hbm, dst_hbm, zeros_hbm, out_hbm,
             src_v, dst_v, hbuf, ebuf, acc, gsem, esem, ssem, isrc, idst):
    _sc_body(hcat_hbm, e_hbm, srccat_hbm, dst_hbm, zeros_hbm, out_hbm,
             src_v, dst_v, hbuf, ebuf, acc, gsem, esem, ssem, isrc, idst)


def kernel(eigen_vectors, edge_index, edge_attr, params):
    x = eigen_vectors
    src = edge_index[0]
    dst = edge_index[1]
    epad = EP - E

    src_p = jnp.concatenate([src, jnp.zeros((epad,), jnp.int32)])
    dst_p = jnp.concatenate([dst, jnp.zeros((epad,), jnp.int32)])
    srccat = jnp.concatenate([src_p, src_p + NP]).reshape(2 * IDX_ROWS, CHUNK)
    dst2d = dst_p.reshape(IDX_ROWS, CHUNK)
    ea_pad = jnp.concatenate(
        [edge_attr, jnp.zeros((epad, D_EDGE), jnp.float32)], axis=0)
    zeros = jnp.zeros((NP, D), jnp.float32)

    xp = jnp.concatenate([x, jnp.zeros((NP - N, D), jnp.float32)], axis=0)
    hcat = jnp.concatenate([xp, -xp], axis=0)
    layers = params['phi']
    e_all = [_edge_lin(ea_pad, p['We'], p['be']) for p in layers]
    for li, p in enumerate(layers[:-1]):
        agg = _sc_pass(hcat, e_all[li], srccat, dst2d, zeros)
        hcat = _mlp(hcat, agg, p['W1'], p['b1'].reshape(1, D),
                    p['W2'], p['b2'].reshape(1, D),
                    p['eps'].reshape(1, 1), out_relu=True)

    p = layers[-1]
    r = params['rho']
    agg = _sc_pass(hcat, e_all[-1], srccat, dst2d, zeros)
    out = _last_and_rho(hcat, agg, p['W1'], p['b1'].reshape(1, D),
                        p['W2'], p['b2'].reshape(1, D), p['eps'].reshape(1, 1),
                        r['W1'], r['b1'].reshape(1, D),
                        r['W2'], r['b2'].reshape(1, D))
    return out[:N]
